# baseline (speedup 1.0000x reference)
import functools

import jax
import jax.numpy as jnp
from jax import lax
from jax.experimental import pallas as pl
from jax.experimental.pallas import tpu as pltpu
from jax.experimental.pallas import tpu_sc as plsc

N = 10000
E = 320000
D = 128
NC = 2
NS = 16
NW = NC * NS
NP = 10240
ROWS_PER_TILE = NP // NS
CHUNK = 128
GROW = 8
GROUPS = 10
TILE_EROWS = GROUPS * GROW
E_PAD = NW * TILE_EROWS * CHUNK
EROWS = E_PAD // CHUNK
ZROWS = 32
ROW_BLK = 1024

_sc_mesh = plsc.VectorSubcoreMesh(core_axis_name="c", subcore_axis_name="s")


@functools.partial(
    pl.kernel,
    mesh=_sc_mesh,
    out_type=[
        jax.ShapeDtypeStruct((NC, NP), jnp.float32),
        jax.ShapeDtypeStruct((NC, NP), jnp.float32),
    ],
    scratch_types=[
        pltpu.VMEM((GROW, CHUNK), jnp.int32),
        pltpu.VMEM((GROW, CHUNK), jnp.int32),
        pltpu.VMEM((CHUNK,), jnp.float32),
        pltpu.VMEM((ROWS_PER_TILE,), jnp.float32),
        pltpu.VMEM_SHARED((NP,), jnp.float32),
        pltpu.VMEM_SHARED((NP,), jnp.float32),
        pltpu.SemaphoreType.DMA,
    ],
)
def _sc_degrees(src_hbm, dst_hbm, ones_hbm, zeros_hbm, dsrc_hbm, ddst_hbm,
                sidx, didx, ones_v, stage, acc_s, acc_d, sem):
    c = lax.axis_index("c")
    s = lax.axis_index("s")
    pltpu.sync_copy(ones_hbm, ones_v)
    pltpu.sync_copy(zeros_hbm, stage)
    row0 = s * ROWS_PER_TILE
    pltpu.sync_copy(stage, acc_s.at[pl.ds(row0, ROWS_PER_TILE)])
    pltpu.sync_copy(stage, acc_d.at[pl.ds(row0, ROWS_PER_TILE)])
    plsc.subcore_barrier()

    erow0 = (c * NS + s) * TILE_EROWS

    def body(g, carry):
        r0 = erow0 + g * GROW
        pltpu.sync_copy(src_hbm.at[pl.ds(r0, GROW)], sidx)
        pltpu.sync_copy(dst_hbm.at[pl.ds(r0, GROW)], didx)
        pend = []
        for j in range(GROW):
            pend.append(
                pltpu.async_copy(ones_v, acc_s.at[sidx.at[j]], sem, add=True))
            pend.append(
                pltpu.async_copy(ones_v, acc_d.at[didx.at[j]], sem, add=True))
        for p in pend:
            p.wait()
        return carry

    lax.fori_loop(0, GROUPS, body, 0)

    plsc.subcore_barrier()
    pltpu.sync_copy(acc_s.at[pl.ds(row0, ROWS_PER_TILE)], stage)
    pltpu.sync_copy(stage, dsrc_hbm.at[c, pl.ds(row0, ROWS_PER_TILE)])
    pltpu.sync_copy(acc_d.at[pl.ds(row0, ROWS_PER_TILE)], stage)
    pltpu.sync_copy(stage, ddst_hbm.at[c, pl.ds(row0, ROWS_PER_TILE)])


@functools.partial(
    pl.kernel,
    mesh=_sc_mesh,
    out_type=jax.ShapeDtypeStruct((NC, NP, D), jnp.float32),
    scratch_types=[
        pltpu.VMEM((GROW, CHUNK), jnp.int32),
        pltpu.VMEM((GROW, CHUNK), jnp.int32),
        pltpu.VMEM((CHUNK, D), jnp.float32),
        pltpu.VMEM((CHUNK, D), jnp.float32),
        pltpu.VMEM((ZROWS, D), jnp.float32),
        pltpu.VMEM((ZROWS, D), jnp.float32),
        pltpu.VMEM_SHARED((NP, D), jnp.float32),
        pltpu.SemaphoreType.DMA,
        pltpu.SemaphoreType.DMA,
        pltpu.SemaphoreType.DMA,
        pltpu.SemaphoreType.DMA,
        pltpu.SemaphoreType.DMA,
        pltpu.SemaphoreType.DMA,
        pltpu.SemaphoreType.DMA,
        pltpu.SemaphoreType.DMA,
    ],
)
def _sc_edge_scatter(h_hbm, src_hbm, dst_hbm, zrows_hbm, out_hbm,
                     sidx, didx, rows0, rows1, st0, st1, acc,
                     gs0, gs1, ss0, ss1, is0, is1, os0, os1):
    c = lax.axis_index("c")
    s = lax.axis_index("s")
    row0 = s * ROWS_PER_TILE
    nz = ROWS_PER_TILE // ZROWS

    pltpu.sync_copy(zrows_hbm, st0)
    zpend = [
        pltpu.async_copy(st0, acc.at[pl.ds(row0 + b * ZROWS, ZROWS)], is0)
        for b in range(nz)
    ]
    for p in zpend:
        p.wait()
    plsc.subcore_barrier()

    erow0 = (c * NS + s) * TILE_EROWS
    B = [rows0, rows1]
    GS = [gs0, gs1]
    SS = [ss0, ss1]

    def body(g, carry):
        r0 = erow0 + g * GROW
        pltpu.sync_copy(src_hbm.at[pl.ds(r0, GROW)], sidx)
        pltpu.sync_copy(dst_hbm.at[pl.ds(r0, GROW)], didx)
        gp = [None, None]
        sp = [None, None]
        gp[0] = pltpu.async_copy(h_hbm.at[sidx.at[0]], B[0], GS[0])
        for j in range(GROW):
            b = j % 2
            gp[b].wait()
            sp[b] = pltpu.async_copy(B[b], acc.at[didx.at[j]], SS[b], add=True)
            if j + 1 < GROW:
                nb = (j + 1) % 2
                if sp[nb] is not None:
                    sp[nb].wait()
                gp[nb] = pltpu.async_copy(h_hbm.at[sidx.at[j + 1]], B[nb], GS[nb])
        for b in range(2):
            if sp[b] is not None:
                sp[b].wait()
        return carry

    lax.fori_loop(0, GROUPS, body, 0)
    plsc.subcore_barrier()

    ST = [st0, st1]
    IS = [is0, is1]
    OS = [os0, os1]
    pin = [None, None]
    pout = [None, None]
    pin[0] = pltpu.async_copy(acc.at[pl.ds(row0, ZROWS)], ST[0], IS[0])
    for b in range(nz):
        cur = b % 2
        if b + 1 < nz:
            nxt = (b + 1) % 2
            if pout[nxt] is not None:
                pout[nxt].wait()
            pin[nxt] = pltpu.async_copy(
                acc.at[pl.ds(row0 + (b + 1) * ZROWS, ZROWS)], ST[nxt], IS[nxt])
        pin[cur].wait()
        pout[cur] = pltpu.async_copy(
            ST[cur], out_hbm.at[c, pl.ds(row0 + b * ZROWS, ZROWS)], OS[cur])
    for p in pout:
        if p is not None:
            p.wait()


def _norm_from_parts(dref):
    deg = dref[0, :] + dref[1, :]
    return lax.rsqrt(jnp.maximum(deg, 1.0))


def _tc_pre_body(x_ref, w_ref, dsrc_ref, o_ref):
    ns = _norm_from_parts(dsrc_ref)
    xw = jnp.dot(x_ref[...], w_ref[...], preferred_element_type=jnp.float32)
    o_ref[...] = xw * ns[:, None]


_tc_pre = pl.pallas_call(
    _tc_pre_body,
    grid=(NP // ROW_BLK,),
    in_specs=[
        pl.BlockSpec((ROW_BLK, D), lambda i: (i, 0)),
        pl.BlockSpec((D, D), lambda i: (0, 0)),
        pl.BlockSpec((NC, ROW_BLK), lambda i: (0, i)),
    ],
    out_specs=pl.BlockSpec((ROW_BLK, D), lambda i: (i, 0)),
    out_shape=jax.ShapeDtypeStruct((NP, D), jnp.float32),
)


def _tc_mid_body(s_ref, ddst_ref, dsrc_ref, b_ref, w_ref, o_ref):
    agg = s_ref[0] + s_ref[1]
    nd = _norm_from_parts(ddst_ref)
    x = jnp.maximum(agg * nd[:, None] + b_ref[...], 0.0)
    ns = _norm_from_parts(dsrc_ref)
    xw = jnp.dot(x, w_ref[...], preferred_element_type=jnp.float32)
    o_ref[...] = xw * ns[:, None]


_tc_mid = pl.pallas_call(
    _tc_mid_body,
    grid=(NP // ROW_BLK,),
    in_specs=[
        pl.BlockSpec((NC, ROW_BLK, D), lambda i: (0, i, 0)),
        pl.BlockSpec((NC, ROW_BLK), lambda i: (0, i)),
        pl.BlockSpec((NC, ROW_BLK), lambda i: (0, i)),
        pl.BlockSpec((1, D), lambda i: (0, 0)),
        pl.BlockSpec((D, D), lambda i: (0, 0)),
    ],
    out_specs=pl.BlockSpec((ROW_BLK, D), lambda i: (i, 0)),
    out_shape=jax.ShapeDtypeStruct((NP, D), jnp.float32),
)


def _tc_final_body(s_ref, ddst_ref, b_ref, wc_ref, bc_ref, o_ref, acc_ref):
    i = pl.program_id(0)
    agg = s_ref[0] + s_ref[1]
    nd = _norm_from_parts(ddst_ref)
    x = jnp.maximum(agg * nd[:, None] + b_ref[...], 0.0)
    rows = lax.broadcasted_iota(jnp.int32, (ROW_BLK, D), 0) + i * ROW_BLK
    x = jnp.where(rows < N, x, 0.0)
    psum = jnp.sum(x, axis=0, keepdims=True)

    @pl.when(i == 0)
    def _():
        acc_ref[...] = psum

    @pl.when(i > 0)
    def _():
        acc_ref[...] = acc_ref[...] + psum

    @pl.when(i == NP // ROW_BLK - 1)
    def _():
        pooled = acc_ref[...] * (1.0 / N)
        o_ref[...] = (
            jnp.dot(pooled, wc_ref[...], preferred_element_type=jnp.float32)
            + bc_ref[...]
        )


_tc_final = pl.pallas_call(
    _tc_final_body,
    grid=(NP // ROW_BLK,),
    in_specs=[
        pl.BlockSpec((NC, ROW_BLK, D), lambda i: (0, i, 0)),
        pl.BlockSpec((NC, ROW_BLK), lambda i: (0, i)),
        pl.BlockSpec((1, D), lambda i: (0, 0)),
        pl.BlockSpec((D, D), lambda i: (0, 0)),
        pl.BlockSpec((1, D), lambda i: (0, 0)),
    ],
    out_specs=pl.BlockSpec((1, D), lambda i: (0, 0)),
    out_shape=jax.ShapeDtypeStruct((1, D), jnp.float32),
    scratch_shapes=[pltpu.VMEM((1, D), jnp.float32)],
)


def kernel(features, edge_index, W1, b1, W2, b2, Wc, bc):
    assert features.shape == (N, D) and edge_index.shape == (2, E)
    d_out = Wc.shape[1]
    feats_p = jnp.pad(features, ((0, NP - N), (0, 0)))
    ones_c = jnp.ones((CHUNK,), jnp.float32)
    zeros_r = jnp.zeros((ROWS_PER_TILE,), jnp.float32)
    zeros_rows = jnp.zeros((ZROWS, D), jnp.float32)

    npad = E_PAD - E
    pad_idx = N + jnp.arange(npad, dtype=jnp.int32) % (NP - N)
    src = jnp.concatenate([edge_index[0], pad_idx]).reshape(EROWS, CHUNK)
    dst = jnp.concatenate([edge_index[1], pad_idx]).reshape(EROWS, CHUNK)

    dsrc, ddst = _sc_degrees(src, dst, ones_c, zeros_r)
    h1 = _tc_pre(feats_p, W1, dsrc)
    s1 = _sc_edge_scatter(h1, src, dst, zeros_rows)
    h2 = _tc_mid(s1, ddst, dsrc, b1.reshape(1, D), W2)
    s2 = _sc_edge_scatter(h2, src, dst, zeros_rows)

    wc_p = jnp.pad(Wc, ((0, 0), (0, D - d_out)))
    bc_p = jnp.pad(bc, (0, D - d_out)).reshape(1, D)
    out = _tc_final(s2, ddst, b2.reshape(1, D), wc_p, bc_p)
    return out[:, :d_out]


# speedup vs baseline: 1.0803x; 1.0803x over previous
import functools

import jax
import jax.numpy as jnp
from jax import lax
from jax.experimental import pallas as pl
from jax.experimental.pallas import tpu as pltpu
from jax.experimental.pallas import tpu_sc as plsc

N = 10000
E = 320000
D = 128
NC = 2
NS = 16
NW = NC * NS
NP = 10240
ROWS_PER_TILE = NP // NS
CHUNK = 128
GROW = 8
GROUPS = 10
TILE_EROWS = GROUPS * GROW
E_PAD = NW * TILE_EROWS * CHUNK
EROWS = E_PAD // CHUNK
ZROWS = 32
ROW_BLK = 1024

_sc_mesh = plsc.VectorSubcoreMesh(core_axis_name="c", subcore_axis_name="s")


@functools.partial(
    pl.kernel,
    mesh=_sc_mesh,
    out_type=[
        jax.ShapeDtypeStruct((NC, NP), jnp.float32),
        jax.ShapeDtypeStruct((NC, NP), jnp.float32),
    ],
    scratch_types=[
        pltpu.VMEM((GROW, CHUNK), jnp.int32),
        pltpu.VMEM((GROW, CHUNK), jnp.int32),
        pltpu.VMEM((CHUNK,), jnp.float32),
        pltpu.VMEM((ROWS_PER_TILE,), jnp.float32),
        pltpu.VMEM_SHARED((NP,), jnp.float32),
        pltpu.VMEM_SHARED((NP,), jnp.float32),
        pltpu.SemaphoreType.DMA,
    ],
)
def _sc_degrees(src_hbm, dst_hbm, ones_hbm, zeros_hbm, dsrc_hbm, ddst_hbm,
                sidx, didx, ones_v, stage, acc_s, acc_d, sem):
    c = lax.axis_index("c")
    s = lax.axis_index("s")
    pltpu.sync_copy(ones_hbm, ones_v)
    pltpu.sync_copy(zeros_hbm, stage)
    row0 = s * ROWS_PER_TILE
    pltpu.sync_copy(stage, acc_s.at[pl.ds(row0, ROWS_PER_TILE)])
    pltpu.sync_copy(stage, acc_d.at[pl.ds(row0, ROWS_PER_TILE)])
    plsc.subcore_barrier()

    erow0 = (c * NS + s) * TILE_EROWS

    def body(g, carry):
        r0 = erow0 + g * GROW
        pltpu.sync_copy(src_hbm.at[pl.ds(r0, GROW)], sidx)
        pltpu.sync_copy(dst_hbm.at[pl.ds(r0, GROW)], didx)
        pend = []
        for j in range(GROW):
            pend.append(
                pltpu.async_copy(ones_v, acc_s.at[sidx.at[j]], sem, add=True))
            pend.append(
                pltpu.async_copy(ones_v, acc_d.at[didx.at[j]], sem, add=True))
        for p in pend:
            p.wait()
        return carry

    lax.fori_loop(0, GROUPS, body, 0)

    plsc.subcore_barrier()
    pltpu.sync_copy(acc_s.at[pl.ds(row0, ROWS_PER_TILE)], stage)
    pltpu.sync_copy(stage, dsrc_hbm.at[c, pl.ds(row0, ROWS_PER_TILE)])
    pltpu.sync_copy(acc_d.at[pl.ds(row0, ROWS_PER_TILE)], stage)
    pltpu.sync_copy(stage, ddst_hbm.at[c, pl.ds(row0, ROWS_PER_TILE)])


@functools.partial(
    pl.kernel,
    mesh=_sc_mesh,
    out_type=jax.ShapeDtypeStruct((NC, NP, D), jnp.float32),
    scratch_types=[
        pltpu.VMEM((GROW, CHUNK), jnp.int32),
        pltpu.VMEM((GROW, CHUNK), jnp.int32),
        pltpu.VMEM((CHUNK, D), jnp.float32),
        pltpu.VMEM((CHUNK, D), jnp.float32),
        pltpu.VMEM((ZROWS, D), jnp.float32),
        pltpu.VMEM((ZROWS, D), jnp.float32),
        pltpu.VMEM_SHARED((NP, D), jnp.float32),
        pltpu.SemaphoreType.DMA,
        pltpu.SemaphoreType.DMA,
        pltpu.SemaphoreType.DMA,
        pltpu.SemaphoreType.DMA,
        pltpu.SemaphoreType.DMA,
        pltpu.SemaphoreType.DMA,
        pltpu.SemaphoreType.DMA,
        pltpu.SemaphoreType.DMA,
    ],
)
def _sc_edge_scatter(h_hbm, src_hbm, dst_hbm, zrows_hbm, out_hbm,
                     sidx, didx, rows0, rows1, st0, st1, acc,
                     gs0, gs1, ss0, ss1, is0, is1, os0, os1):
    c = lax.axis_index("c")
    s = lax.axis_index("s")
    row0 = s * ROWS_PER_TILE
    nz = ROWS_PER_TILE // ZROWS

    pltpu.sync_copy(zrows_hbm, st0)
    zpend = [
        pltpu.async_copy(st0, acc.at[pl.ds(row0 + b * ZROWS, ZROWS)], is0)
        for b in range(nz)
    ]
    for p in zpend:
        p.wait()
    plsc.subcore_barrier()

    erow0 = (c * NS + s) * TILE_EROWS
    B = [rows0, rows1]
    GS = [gs0, gs1]
    SS = [ss0, ss1]

    def body(g, carry):
        r0 = erow0 + g * GROW
        pltpu.sync_copy(src_hbm.at[pl.ds(r0, GROW)], sidx)
        pltpu.sync_copy(dst_hbm.at[pl.ds(r0, GROW)], didx)
        pend = pltpu.async_copy(h_hbm.at[sidx.at[0]], B[0], GS[0])
        for j in range(GROW):
            cur = B[j % 2]
            if j + 1 < GROW:
                nb = (j + 1) % 2
                pend_next = pltpu.async_copy(h_hbm.at[sidx.at[j + 1]], B[nb], GS[nb])
            pend.wait()
            pltpu.sync_copy(cur, acc.at[didx.at[j]], add=True)
            if j + 1 < GROW:
                pend = pend_next
        return carry

    lax.fori_loop(0, GROUPS, body, 0)
    plsc.subcore_barrier()

    ST = [st0, st1]
    IS = [is0, is1]
    OS = [os0, os1]
    pin = [None, None]
    pout = [None, None]
    pin[0] = pltpu.async_copy(acc.at[pl.ds(row0, ZROWS)], ST[0], IS[0])
    for b in range(nz):
        cur = b % 2
        if b + 1 < nz:
            nxt = (b + 1) % 2
            if pout[nxt] is not None:
                pout[nxt].wait()
            pin[nxt] = pltpu.async_copy(
                acc.at[pl.ds(row0 + (b + 1) * ZROWS, ZROWS)], ST[nxt], IS[nxt])
        pin[cur].wait()
        pout[cur] = pltpu.async_copy(
            ST[cur], out_hbm.at[c, pl.ds(row0 + b * ZROWS, ZROWS)], OS[cur])
    for p in pout:
        if p is not None:
            p.wait()


def _norm_from_parts(dref):
    deg = dref[0, :] + dref[1, :]
    return lax.rsqrt(jnp.maximum(deg, 1.0))


def _tc_pre_body(x_ref, w_ref, dsrc_ref, o_ref):
    ns = _norm_from_parts(dsrc_ref)
    xw = jnp.dot(x_ref[...], w_ref[...], preferred_element_type=jnp.float32)
    o_ref[...] = xw * ns[:, None]


_tc_pre = pl.pallas_call(
    _tc_pre_body,
    grid=(NP // ROW_BLK,),
    in_specs=[
        pl.BlockSpec((ROW_BLK, D), lambda i: (i, 0)),
        pl.BlockSpec((D, D), lambda i: (0, 0)),
        pl.BlockSpec((NC, ROW_BLK), lambda i: (0, i)),
    ],
    out_specs=pl.BlockSpec((ROW_BLK, D), lambda i: (i, 0)),
    out_shape=jax.ShapeDtypeStruct((NP, D), jnp.float32),
)


def _tc_mid_body(s_ref, ddst_ref, dsrc_ref, b_ref, w_ref, o_ref):
    agg = s_ref[0] + s_ref[1]
    nd = _norm_from_parts(ddst_ref)
    x = jnp.maximum(agg * nd[:, None] + b_ref[...], 0.0)
    ns = _norm_from_parts(dsrc_ref)
    xw = jnp.dot(x, w_ref[...], preferred_element_type=jnp.float32)
    o_ref[...] = xw * ns[:, None]


_tc_mid = pl.pallas_call(
    _tc_mid_body,
    grid=(NP // ROW_BLK,),
    in_specs=[
        pl.BlockSpec((NC, ROW_BLK, D), lambda i: (0, i, 0)),
        pl.BlockSpec((NC, ROW_BLK), lambda i: (0, i)),
        pl.BlockSpec((NC, ROW_BLK), lambda i: (0, i)),
        pl.BlockSpec((1, D), lambda i: (0, 0)),
        pl.BlockSpec((D, D), lambda i: (0, 0)),
    ],
    out_specs=pl.BlockSpec((ROW_BLK, D), lambda i: (i, 0)),
    out_shape=jax.ShapeDtypeStruct((NP, D), jnp.float32),
)


def _tc_final_body(s_ref, ddst_ref, b_ref, wc_ref, bc_ref, o_ref, acc_ref):
    i = pl.program_id(0)
    agg = s_ref[0] + s_ref[1]
    nd = _norm_from_parts(ddst_ref)
    x = jnp.maximum(agg * nd[:, None] + b_ref[...], 0.0)
    rows = lax.broadcasted_iota(jnp.int32, (ROW_BLK, D), 0) + i * ROW_BLK
    x = jnp.where(rows < N, x, 0.0)
    psum = jnp.sum(x, axis=0, keepdims=True)

    @pl.when(i == 0)
    def _():
        acc_ref[...] = psum

    @pl.when(i > 0)
    def _():
        acc_ref[...] = acc_ref[...] + psum

    @pl.when(i == NP // ROW_BLK - 1)
    def _():
        pooled = acc_ref[...] * (1.0 / N)
        o_ref[...] = (
            jnp.dot(pooled, wc_ref[...], preferred_element_type=jnp.float32)
            + bc_ref[...]
        )


_tc_final = pl.pallas_call(
    _tc_final_body,
    grid=(NP // ROW_BLK,),
    in_specs=[
        pl.BlockSpec((NC, ROW_BLK, D), lambda i: (0, i, 0)),
        pl.BlockSpec((NC, ROW_BLK), lambda i: (0, i)),
        pl.BlockSpec((1, D), lambda i: (0, 0)),
        pl.BlockSpec((D, D), lambda i: (0, 0)),
        pl.BlockSpec((1, D), lambda i: (0, 0)),
    ],
    out_specs=pl.BlockSpec((1, D), lambda i: (0, 0)),
    out_shape=jax.ShapeDtypeStruct((1, D), jnp.float32),
    scratch_shapes=[pltpu.VMEM((1, D), jnp.float32)],
)


def kernel(features, edge_index, W1, b1, W2, b2, Wc, bc):
    assert features.shape == (N, D) and edge_index.shape == (2, E)
    d_out = Wc.shape[1]
    feats_p = jnp.pad(features, ((0, NP - N), (0, 0)))
    ones_c = jnp.ones((CHUNK,), jnp.float32)
    zeros_r = jnp.zeros((ROWS_PER_TILE,), jnp.float32)
    zeros_rows = jnp.zeros((ZROWS, D), jnp.float32)

    npad = E_PAD - E
    pad_idx = N + jnp.arange(npad, dtype=jnp.int32) % (NP - N)
    src = jnp.concatenate([edge_index[0], pad_idx]).reshape(EROWS, CHUNK)
    dst = jnp.concatenate([edge_index[1], pad_idx]).reshape(EROWS, CHUNK)

    dsrc, ddst = _sc_degrees(src, dst, ones_c, zeros_r)
    h1 = _tc_pre(feats_p, W1, dsrc)
    s1 = _sc_edge_scatter(h1, src, dst, zeros_rows)
    h2 = _tc_mid(s1, ddst, dsrc, b1.reshape(1, D), W2)
    s2 = _sc_edge_scatter(h2, src, dst, zeros_rows)

    wc_p = jnp.pad(Wc, ((0, 0), (0, D - d_out)))
    bc_p = jnp.pad(bc, (0, D - d_out)).reshape(1, D)
    out = _tc_final(s2, ddst, b2.reshape(1, D), wc_p, bc_p)
    return out[:, :d_out]


# speedup vs baseline: 1.1743x; 1.0871x over previous
import functools

import jax
import jax.numpy as jnp
from jax import lax
from jax.experimental import pallas as pl
from jax.experimental.pallas import tpu as pltpu
from jax.experimental.pallas import tpu_sc as plsc

N = 10000
E = 320000
D = 128
NC = 2
NS = 16
NW = NC * NS
NP = 10240
ROWS_PER_TILE = NP // NS
CHUNK = 128
GROW = 16
GROUPS = 5
TILE_EROWS = GROUPS * GROW
E_PAD = NW * TILE_EROWS * CHUNK
EROWS = E_PAD // CHUNK
ZROWS = 32
ROW_BLK = 1024

_sc_mesh = plsc.VectorSubcoreMesh(core_axis_name="c", subcore_axis_name="s")


@functools.partial(
    pl.kernel,
    mesh=_sc_mesh,
    out_type=[
        jax.ShapeDtypeStruct((NC, NP), jnp.float32),
        jax.ShapeDtypeStruct((NC, NP), jnp.float32),
    ],
    scratch_types=[
        pltpu.VMEM((GROW, CHUNK), jnp.int32),
        pltpu.VMEM((GROW, CHUNK), jnp.int32),
        pltpu.VMEM((CHUNK,), jnp.float32),
        pltpu.VMEM((ROWS_PER_TILE,), jnp.float32),
        pltpu.VMEM_SHARED((NP,), jnp.float32),
        pltpu.VMEM_SHARED((NP,), jnp.float32),
        pltpu.SemaphoreType.DMA,
    ],
)
def _sc_degrees(src_hbm, dst_hbm, ones_hbm, zeros_hbm, dsrc_hbm, ddst_hbm,
                sidx, didx, ones_v, stage, acc_s, acc_d, sem):
    c = lax.axis_index("c")
    s = lax.axis_index("s")
    pltpu.sync_copy(ones_hbm, ones_v)
    pltpu.sync_copy(zeros_hbm, stage)
    row0 = s * ROWS_PER_TILE
    pltpu.sync_copy(stage, acc_s.at[pl.ds(row0, ROWS_PER_TILE)])
    pltpu.sync_copy(stage, acc_d.at[pl.ds(row0, ROWS_PER_TILE)])
    plsc.subcore_barrier()

    erow0 = (c * NS + s) * TILE_EROWS

    def body(g, carry):
        r0 = erow0 + g * GROW
        pltpu.sync_copy(src_hbm.at[pl.ds(r0, GROW)], sidx)
        pltpu.sync_copy(dst_hbm.at[pl.ds(r0, GROW)], didx)
        pend = []
        for j in range(GROW):
            pend.append(
                pltpu.async_copy(ones_v, acc_s.at[sidx.at[j]], sem, add=True))
            pend.append(
                pltpu.async_copy(ones_v, acc_d.at[didx.at[j]], sem, add=True))
        for p in pend:
            p.wait()
        return carry

    lax.fori_loop(0, GROUPS, body, 0)

    plsc.subcore_barrier()
    pltpu.sync_copy(acc_s.at[pl.ds(row0, ROWS_PER_TILE)], stage)
    pltpu.sync_copy(stage, dsrc_hbm.at[c, pl.ds(row0, ROWS_PER_TILE)])
    pltpu.sync_copy(acc_d.at[pl.ds(row0, ROWS_PER_TILE)], stage)
    pltpu.sync_copy(stage, ddst_hbm.at[c, pl.ds(row0, ROWS_PER_TILE)])


@functools.partial(
    pl.kernel,
    mesh=_sc_mesh,
    out_type=jax.ShapeDtypeStruct((NC, NP, D), jnp.float32),
    scratch_types=[
        pltpu.VMEM((GROW, CHUNK), jnp.int32),
        pltpu.VMEM((GROW, CHUNK), jnp.int32),
        pltpu.VMEM((CHUNK, D), jnp.float32),
        pltpu.VMEM((CHUNK, D), jnp.float32),
        pltpu.VMEM((ZROWS, D), jnp.float32),
        pltpu.VMEM((ZROWS, D), jnp.float32),
        pltpu.VMEM_SHARED((NP, D), jnp.float32),
        pltpu.SemaphoreType.DMA,
        pltpu.SemaphoreType.DMA,
        pltpu.SemaphoreType.DMA,
        pltpu.SemaphoreType.DMA,
        pltpu.SemaphoreType.DMA,
        pltpu.SemaphoreType.DMA,
        pltpu.SemaphoreType.DMA,
        pltpu.SemaphoreType.DMA,
    ],
)
def _sc_edge_scatter(h_hbm, src_hbm, dst_hbm, zrows_hbm, out_hbm,
                     sidx, didx, rows0, rows1, st0, st1, acc,
                     gs0, gs1, ss0, ss1, is0, is1, os0, os1):
    c = lax.axis_index("c")
    s = lax.axis_index("s")
    row0 = s * ROWS_PER_TILE
    nz = ROWS_PER_TILE // ZROWS

    pltpu.sync_copy(zrows_hbm, st0)
    zpend = [
        pltpu.async_copy(st0, acc.at[pl.ds(row0 + b * ZROWS, ZROWS)], is0)
        for b in range(nz)
    ]
    for p in zpend:
        p.wait()
    plsc.subcore_barrier()

    erow0 = (c * NS + s) * TILE_EROWS
    B = [rows0, rows1]
    GS = [gs0, gs1]
    SS = [ss0, ss1]

    def body(g, carry):
        r0 = erow0 + g * GROW
        pltpu.sync_copy(src_hbm.at[pl.ds(r0, GROW)], sidx)
        pltpu.sync_copy(dst_hbm.at[pl.ds(r0, GROW)], didx)
        pend = pltpu.async_copy(h_hbm.at[sidx.at[0]], B[0], GS[0])
        for j in range(GROW):
            cur = B[j % 2]
            if j + 1 < GROW:
                nb = (j + 1) % 2
                pend_next = pltpu.async_copy(h_hbm.at[sidx.at[j + 1]], B[nb], GS[nb])
            pend.wait()
            pltpu.sync_copy(cur, acc.at[didx.at[j]], add=True)
            if j + 1 < GROW:
                pend = pend_next
        return carry

    lax.fori_loop(0, GROUPS, body, 0)
    plsc.subcore_barrier()

    ST = [st0, st1]
    IS = [is0, is1]
    OS = [os0, os1]
    pin = [None, None]
    pout = [None, None]
    pin[0] = pltpu.async_copy(acc.at[pl.ds(row0, ZROWS)], ST[0], IS[0])
    for b in range(nz):
        cur = b % 2
        if b + 1 < nz:
            nxt = (b + 1) % 2
            if pout[nxt] is not None:
                pout[nxt].wait()
            pin[nxt] = pltpu.async_copy(
                acc.at[pl.ds(row0 + (b + 1) * ZROWS, ZROWS)], ST[nxt], IS[nxt])
        pin[cur].wait()
        pout[cur] = pltpu.async_copy(
            ST[cur], out_hbm.at[c, pl.ds(row0 + b * ZROWS, ZROWS)], OS[cur])
    for p in pout:
        if p is not None:
            p.wait()


def _norm_from_parts(dref):
    deg = dref[0, :] + dref[1, :]
    return lax.rsqrt(jnp.maximum(deg, 1.0))


def _tc_pre_body(x_ref, w_ref, dsrc_ref, o_ref):
    ns = _norm_from_parts(dsrc_ref)
    xw = jnp.dot(x_ref[...], w_ref[...], preferred_element_type=jnp.float32)
    o_ref[...] = xw * ns[:, None]


_tc_pre = pl.pallas_call(
    _tc_pre_body,
    grid=(NP // ROW_BLK,),
    in_specs=[
        pl.BlockSpec((ROW_BLK, D), lambda i: (i, 0)),
        pl.BlockSpec((D, D), lambda i: (0, 0)),
        pl.BlockSpec((NC, ROW_BLK), lambda i: (0, i)),
    ],
    out_specs=pl.BlockSpec((ROW_BLK, D), lambda i: (i, 0)),
    out_shape=jax.ShapeDtypeStruct((NP, D), jnp.float32),
)


def _tc_mid_body(s_ref, ddst_ref, dsrc_ref, b_ref, w_ref, o_ref):
    agg = s_ref[0] + s_ref[1]
    nd = _norm_from_parts(ddst_ref)
    x = jnp.maximum(agg * nd[:, None] + b_ref[...], 0.0)
    ns = _norm_from_parts(dsrc_ref)
    xw = jnp.dot(x, w_ref[...], preferred_element_type=jnp.float32)
    o_ref[...] = xw * ns[:, None]


_tc_mid = pl.pallas_call(
    _tc_mid_body,
    grid=(NP // ROW_BLK,),
    in_specs=[
        pl.BlockSpec((NC, ROW_BLK, D), lambda i: (0, i, 0)),
        pl.BlockSpec((NC, ROW_BLK), lambda i: (0, i)),
        pl.BlockSpec((NC, ROW_BLK), lambda i: (0, i)),
        pl.BlockSpec((1, D), lambda i: (0, 0)),
        pl.BlockSpec((D, D), lambda i: (0, 0)),
    ],
    out_specs=pl.BlockSpec((ROW_BLK, D), lambda i: (i, 0)),
    out_shape=jax.ShapeDtypeStruct((NP, D), jnp.float32),
)


def _tc_final_body(s_ref, ddst_ref, b_ref, wc_ref, bc_ref, o_ref, acc_ref):
    i = pl.program_id(0)
    agg = s_ref[0] + s_ref[1]
    nd = _norm_from_parts(ddst_ref)
    x = jnp.maximum(agg * nd[:, None] + b_ref[...], 0.0)
    rows = lax.broadcasted_iota(jnp.int32, (ROW_BLK, D), 0) + i * ROW_BLK
    x = jnp.where(rows < N, x, 0.0)
    psum = jnp.sum(x, axis=0, keepdims=True)

    @pl.when(i == 0)
    def _():
        acc_ref[...] = psum

    @pl.when(i > 0)
    def _():
        acc_ref[...] = acc_ref[...] + psum

    @pl.when(i == NP // ROW_BLK - 1)
    def _():
        pooled = acc_ref[...] * (1.0 / N)
        o_ref[...] = (
            jnp.dot(pooled, wc_ref[...], preferred_element_type=jnp.float32)
            + bc_ref[...]
        )


_tc_final = pl.pallas_call(
    _tc_final_body,
    grid=(NP // ROW_BLK,),
    in_specs=[
        pl.BlockSpec((NC, ROW_BLK, D), lambda i: (0, i, 0)),
        pl.BlockSpec((NC, ROW_BLK), lambda i: (0, i)),
        pl.BlockSpec((1, D), lambda i: (0, 0)),
        pl.BlockSpec((D, D), lambda i: (0, 0)),
        pl.BlockSpec((1, D), lambda i: (0, 0)),
    ],
    out_specs=pl.BlockSpec((1, D), lambda i: (0, 0)),
    out_shape=jax.ShapeDtypeStruct((1, D), jnp.float32),
    scratch_shapes=[pltpu.VMEM((1, D), jnp.float32)],
)


def kernel(features, edge_index, W1, b1, W2, b2, Wc, bc):
    assert features.shape == (N, D) and edge_index.shape == (2, E)
    d_out = Wc.shape[1]
    feats_p = jnp.pad(features, ((0, NP - N), (0, 0)))
    ones_c = jnp.ones((CHUNK,), jnp.float32)
    zeros_r = jnp.zeros((ROWS_PER_TILE,), jnp.float32)
    zeros_rows = jnp.zeros((ZROWS, D), jnp.float32)

    npad = E_PAD - E
    pad_idx = N + jnp.arange(npad, dtype=jnp.int32) % (NP - N)
    src = jnp.concatenate([edge_index[0], pad_idx]).reshape(EROWS, CHUNK)
    dst = jnp.concatenate([edge_index[1], pad_idx]).reshape(EROWS, CHUNK)

    dsrc, ddst = _sc_degrees(src, dst, ones_c, zeros_r)
    h1 = _tc_pre(feats_p, W1, dsrc)
    s1 = _sc_edge_scatter(h1, src, dst, zeros_rows)
    h2 = _tc_mid(s1, ddst, dsrc, b1.reshape(1, D), W2)
    s2 = _sc_edge_scatter(h2, src, dst, zeros_rows)

    wc_p = jnp.pad(Wc, ((0, 0), (0, D - d_out)))
    bc_p = jnp.pad(bc, (0, D - d_out)).reshape(1, D)
    out = _tc_final(s2, ddst, b2.reshape(1, D), wc_p, bc_p)
    return out[:, :d_out]
